# Initial kernel scaffold; baseline (speedup 1.0000x reference)
import jax
import jax.numpy as jnp
from jax.experimental import pallas as pl


def kernel(x, table_xy, table_xz, table_yz):
    raise NotImplementedError("write your pallas kernel here")


# speedup vs baseline: 59.9635x; 59.9635x over previous
import functools

import jax
import jax.numpy as jnp
import numpy as np
from jax import lax
from jax.experimental import pallas as pl
from jax.experimental.pallas import tpu as pltpu
from jax.experimental.pallas import tpu_sc as plsc

_RES = 512
_FEAT = 16
_MASK = _RES * _RES - 1
_PRIME1 = np.int32(np.uint32(2654435761).astype(np.int64) - (1 << 32))

_B = 1048576
_L = 16
_NC, _NS = 2, 16
_NW = _NC * _NS
_P = 512
_PW = _B // _NW
_NCHUNK = _PW // _P
_GSUB = 128

_PLANES = ((0, 1), (0, 2), (1, 2))


def _tri_embed_body(x0_hbm, x1_hbm, x2_hbm, t0_hbm, t1_hbm, t2_hbm, out_hbm,
                    x_v, w_v, i_v, r_v, out_v, sem):
    wid = lax.axis_index("s") * _NC + lax.axis_index("c")
    base = wid * _PW
    tables = (t0_hbm, t1_hbm, t2_hbm)

    def chunk_body(c, carry):
        p0 = base + c * _P
        for xh, xv in ((x0_hbm, x_v[0]), (x1_hbm, x_v[1]), (x2_hbm, x_v[2])):
            pltpu.sync_copy(xh.at[pl.ds(p0, _P)], xv)

        def hash_body(g, carry):
            s = pl.ds(g * _L, _L)
            cell = []
            for k in range(3):
                f = x_v[k][s] * float(_RES)
                i = f.astype(jnp.int32)
                i = jnp.minimum(jnp.maximum(i, 0), _RES - 1)
                w_v[k][s] = f - i.astype(jnp.float32)
                cell.append(i)
            for p_i, (a, b) in enumerate(_PLANES):
                ia, ib = cell[a], cell[b]
                t0 = ib * _PRIME1
                t1 = t0 + _PRIME1
                i_v[p_i][0][s] = (ia ^ t0) & _MASK
                i_v[p_i][1][s] = (ia ^ t1) & _MASK
                ia1 = ia + 1
                i_v[p_i][2][s] = (ia1 ^ t0) & _MASK
                i_v[p_i][3][s] = (ia1 ^ t1) & _MASK
            return carry

        lax.fori_loop(0, _P // _L, hash_body, 0, unroll=2)

        for p_i, (a, b) in enumerate(_PLANES):
            table = tables[p_i]

            def gather_body(j, carry, p_i=p_i, table=table):
                js = pl.ds(j * _GSUB, _GSUB)
                hs = [
                    pltpu.async_copy(table.at[i_v[p_i][k].at[js]],
                                     r_v[k].at[js], sem)
                    for k in range(4)
                ]
                for h in hs:
                    h.wait()
                return carry

            lax.fori_loop(0, _P // _GSUB, gather_body, 0)

            def interp_body(g, carry, p_i=p_i, a=a, b=b):
                s = pl.ds(g * _L, _L)
                wa16 = w_v[a][s]
                wb16 = w_v[b][s]
                for k in range(_L):
                    p = g * _L + k
                    wa = jnp.zeros((_L,), jnp.float32) + wa16[k]
                    wb = jnp.zeros((_L,), jnp.float32) + wb16[k]
                    e00 = r_v[0][p]
                    e01 = r_v[1][p]
                    e10 = r_v[2][p]
                    e11 = r_v[3][p]
                    lo = e00 + wa * (e10 - e00)
                    hi = e01 + wa * (e11 - e01)
                    out_v[p, pl.ds(p_i * _FEAT, _FEAT)] = lo + wb * (hi - lo)
                return carry

            lax.fori_loop(0, _P // _L, interp_body, 0)

        pltpu.sync_copy(out_v, out_hbm.at[pl.ds(p0, _P)])
        return carry

    lax.fori_loop(0, _NCHUNK, chunk_body, 0)


@jax.jit
def kernel(x, table_xy, table_xz, table_yz):
    x0 = x[:, 0]
    x1 = x[:, 1]
    x2 = x[:, 2]
    scratch = dict(
        x_v=[pltpu.VMEM((_P,), jnp.float32) for _ in range(3)],
        w_v=[pltpu.VMEM((_P,), jnp.float32) for _ in range(3)],
        i_v=[[pltpu.VMEM((_P,), jnp.int32) for _ in range(4)]
             for _ in range(3)],
        r_v=[pltpu.VMEM((_P, _FEAT), jnp.float32) for _ in range(4)],
        out_v=pltpu.VMEM((_P, 3 * _FEAT), jnp.float32),
        sem=pltpu.SemaphoreType.DMA,
    )
    run = pl.kernel(
        _tri_embed_body,
        out_type=jax.ShapeDtypeStruct((_B, 3 * _FEAT), jnp.float32),
        mesh=plsc.VectorSubcoreMesh(core_axis_name="c", subcore_axis_name="s",
                                    num_cores=_NC, num_subcores=_NS),
        scratch_types=scratch,
        compiler_params=pltpu.CompilerParams(use_tc_tiling_on_sc=False),
    )
    return run(x0, x1, x2, table_xy, table_xz, table_yz)


# speedup vs baseline: 83.8107x; 1.3977x over previous
import functools

import jax
import jax.numpy as jnp
import numpy as np
from jax import lax
from jax.experimental import pallas as pl
from jax.experimental.pallas import tpu as pltpu
from jax.experimental.pallas import tpu_sc as plsc

_RES = 512
_FEAT = 16
_MASK = _RES * _RES - 1
_PRIME1 = np.int32(np.uint32(2654435761).astype(np.int64) - (1 << 32))

_B = 1048576
_L = 16
_NC, _NS = 2, 16
_NW = _NC * _NS
_P = 512
_PW = _B // _NW
_NCHUNK = _PW // _P
_GSUB = 128

_PLANES = ((0, 1), (0, 2), (1, 2))


def _tri_embed_body(x0_hbm, x1_hbm, x2_hbm, t0_hbm, t1_hbm, t2_hbm, out_hbm,
                    x_v, w_v, i_v, r_v, out_v, sems):
    wid = lax.axis_index("s") * _NC + lax.axis_index("c")
    base = wid * _PW
    tables = (t0_hbm, t1_hbm, t2_hbm)

    def load_and_hash(c, ci):
        p0 = base + c * _P
        for xh, xv in ((x0_hbm, x_v[0]), (x1_hbm, x_v[1]), (x2_hbm, x_v[2])):
            pltpu.sync_copy(xh.at[pl.ds(p0, _P)], xv)

        def hash_body(g, carry):
            s = pl.ds(g * _L, _L)
            cell = []
            for k in range(3):
                f = x_v[k][s] * float(_RES)
                i = f.astype(jnp.int32)
                i = jnp.minimum(jnp.maximum(i, 0), _RES - 1)
                w_v[ci][k][s] = f - i.astype(jnp.float32)
                cell.append(i)
            for p_i, (a, b) in enumerate(_PLANES):
                ia, ib = cell[a], cell[b]
                t0 = ib * _PRIME1
                t1 = t0 + _PRIME1
                i_v[ci][p_i][0][s] = (ia ^ t0) & _MASK
                i_v[ci][p_i][1][s] = (ia ^ t1) & _MASK
                ia1 = ia + 1
                i_v[ci][p_i][2][s] = (ia1 ^ t0) & _MASK
                i_v[ci][p_i][3][s] = (ia1 ^ t1) & _MASK
            return carry

        lax.fori_loop(0, _P // _L, hash_body, 0, unroll=2)

    def issue(ci, p_i, b):
        table = tables[p_i]
        for j in range(_P // _GSUB):
            js = pl.ds(j * _GSUB, _GSUB)
            for k in range(4):
                pltpu.async_copy(table.at[i_v[ci][p_i][k].at[js]],
                                 r_v[b][k].at[js], sems[b])

    def drain(b):
        for k in range(4):
            pltpu.make_async_copy(t0_hbm.at[pl.ds(0, _P)], r_v[b][k],
                                  sems[b]).wait()

    def interp(ci, p_i, b):
        a, bb = _PLANES[p_i]

        def interp_body(g, carry):
            s = pl.ds(g * _L, _L)
            wa16 = w_v[ci][a][s]
            wb16 = w_v[ci][bb][s]
            for k in range(_L):
                p = g * _L + k
                wa = jnp.zeros((_L,), jnp.float32) + wa16[k]
                wb = jnp.zeros((_L,), jnp.float32) + wb16[k]
                e00 = r_v[b][0][p]
                e01 = r_v[b][1][p]
                e10 = r_v[b][2][p]
                e11 = r_v[b][3][p]
                lo = e00 + wa * (e10 - e00)
                hi = e01 + wa * (e11 - e01)
                out_v[p, pl.ds(p_i * _FEAT, _FEAT)] = lo + wb * (hi - lo)
            return carry

        lax.fori_loop(0, _P // _L, interp_body, 0)

    def unit(g, j, last=False):
        c = 2 * g + (j // 3)
        p_i = j % 3
        b = j % 2
        ci = j // 3
        if not last:
            jn = j + 1
            cn = 2 * g + (jn // 3)
            if jn % 3 == 0:
                load_and_hash(jnp.minimum(cn, _NCHUNK - 1), (jn // 3) % 2)
            issue((jn // 3) % 2, jn % 3, jn % 2)
        drain(b)
        interp(ci, p_i, b)
        if p_i == 2:
            pltpu.sync_copy(out_v, out_hbm.at[pl.ds(base + c * _P, _P)])

    load_and_hash(0, 0)
    issue(0, 0, 0)

    def group(g, carry):
        for j in range(6):
            unit(g, j)
        return carry

    lax.fori_loop(0, _NCHUNK // 2, group, 0)
    drain(0)


@jax.jit
def kernel(x, table_xy, table_xz, table_yz):
    x0 = x[:, 0]
    x1 = x[:, 1]
    x2 = x[:, 2]
    scratch = dict(
        x_v=[pltpu.VMEM((_P,), jnp.float32) for _ in range(3)],
        w_v=[[pltpu.VMEM((_P,), jnp.float32) for _ in range(3)]
             for _ in range(2)],
        i_v=[[[pltpu.VMEM((_P,), jnp.int32) for _ in range(4)]
              for _ in range(3)] for _ in range(2)],
        r_v=[[pltpu.VMEM((_P, _FEAT), jnp.float32) for _ in range(4)]
             for _ in range(2)],
        out_v=pltpu.VMEM((_P, 3 * _FEAT), jnp.float32),
        sems=[pltpu.SemaphoreType.DMA for _ in range(2)],
    )
    run = pl.kernel(
        _tri_embed_body,
        out_type=jax.ShapeDtypeStruct((_B, 3 * _FEAT), jnp.float32),
        mesh=plsc.VectorSubcoreMesh(core_axis_name="c", subcore_axis_name="s",
                                    num_cores=_NC, num_subcores=_NS),
        scratch_types=scratch,
        compiler_params=pltpu.CompilerParams(use_tc_tiling_on_sc=False),
    )
    return run(x0, x1, x2, table_xy, table_xz, table_yz)


# speedup vs baseline: 124.3376x; 1.4836x over previous
import functools

import jax
import jax.numpy as jnp
import numpy as np
from jax import lax
from jax.experimental import pallas as pl
from jax.experimental.pallas import tpu as pltpu
from jax.experimental.pallas import tpu_sc as plsc

_RES = 512
_FEAT = 16
_MASK = _RES * _RES - 1
_PRIME1 = np.int32(np.uint32(2654435761).astype(np.int64) - (1 << 32))

_B = 1048576
_L = 16
_NC, _NS = 2, 16
_NW = _NC * _NS
_P = 512
_PW = _B // _NW
_NCHUNK = _PW // _P
_GSUB = 128

_PLANES = ((0, 1), (0, 2), (1, 2))


def _tri_embed_body(x0_hbm, x1_hbm, x2_hbm, t0_hbm, t1_hbm, t2_hbm, out_hbm,
                    x_v, w_v, i_v, r_v, out_v, sems):
    wid = lax.axis_index("s") * _NC + lax.axis_index("c")
    base = wid * _PW
    tables = (t0_hbm, t1_hbm, t2_hbm)

    def load_and_hash(c, ci):
        p0 = base + c * _P
        for xh, xv in ((x0_hbm, x_v[0]), (x1_hbm, x_v[1]), (x2_hbm, x_v[2])):
            pltpu.sync_copy(xh.at[pl.ds(p0, _P)], xv)

        @plsc.parallel_loop(0, _P // _L, unroll=2)
        def hash_body(g):
            s = pl.ds(g * _L, _L)
            cell = []
            for k in range(3):
                f = x_v[k][s] * float(_RES)
                i = f.astype(jnp.int32)
                i = jnp.minimum(jnp.maximum(i, 0), _RES - 1)
                w_v[ci][k][s] = f - i.astype(jnp.float32)
                cell.append(i)
            for p_i, (a, b) in enumerate(_PLANES):
                ia, ib = cell[a], cell[b]
                t0 = ib * _PRIME1
                t1 = t0 + _PRIME1
                i_v[ci][p_i][0][s] = (ia ^ t0) & _MASK
                i_v[ci][p_i][1][s] = (ia ^ t1) & _MASK
                ia1 = ia + 1
                i_v[ci][p_i][2][s] = (ia1 ^ t0) & _MASK
                i_v[ci][p_i][3][s] = (ia1 ^ t1) & _MASK

    def issue(ci, p_i, b):
        table = tables[p_i]
        for j in range(_P // _GSUB):
            js = pl.ds(j * _GSUB, _GSUB)
            for k in range(4):
                pltpu.async_copy(table.at[i_v[ci][p_i][k].at[js]],
                                 r_v[b][k].at[js], sems[b])

    def drain(b):
        for k in range(4):
            pltpu.make_async_copy(t0_hbm.at[pl.ds(0, _P)], r_v[b][k],
                                  sems[b]).wait()

    def interp(ci, p_i, b):
        a, bb = _PLANES[p_i]

        @plsc.parallel_loop(0, _P // _L)
        def interp_body(g):
            s = pl.ds(g * _L, _L)
            wa16 = w_v[ci][a][s]
            wb16 = w_v[ci][bb][s]
            for k in range(_L):
                p = g * _L + k
                wa = jnp.zeros((_L,), jnp.float32) + wa16[k]
                wb = jnp.zeros((_L,), jnp.float32) + wb16[k]
                e00 = r_v[b][0][p]
                e01 = r_v[b][1][p]
                e10 = r_v[b][2][p]
                e11 = r_v[b][3][p]
                lo = e00 + wa * (e10 - e00)
                hi = e01 + wa * (e11 - e01)
                out_v[p, pl.ds(p_i * _FEAT, _FEAT)] = lo + wb * (hi - lo)

    def unit(g, j, last=False):
        c = 2 * g + (j // 3)
        p_i = j % 3
        b = j % 2
        ci = j // 3
        if not last:
            jn = j + 1
            cn = 2 * g + (jn // 3)
            if jn % 3 == 0:
                load_and_hash(jnp.minimum(cn, _NCHUNK - 1), (jn // 3) % 2)
            issue((jn // 3) % 2, jn % 3, jn % 2)
        drain(b)
        interp(ci, p_i, b)
        if p_i == 2:
            pltpu.sync_copy(out_v, out_hbm.at[pl.ds(base + c * _P, _P)])

    load_and_hash(0, 0)
    issue(0, 0, 0)

    def group(g, carry):
        for j in range(6):
            unit(g, j)
        return carry

    lax.fori_loop(0, _NCHUNK // 2, group, 0)
    drain(0)


@jax.jit
def kernel(x, table_xy, table_xz, table_yz):
    x0 = x[:, 0]
    x1 = x[:, 1]
    x2 = x[:, 2]
    scratch = dict(
        x_v=[pltpu.VMEM((_P,), jnp.float32) for _ in range(3)],
        w_v=[[pltpu.VMEM((_P,), jnp.float32) for _ in range(3)]
             for _ in range(2)],
        i_v=[[[pltpu.VMEM((_P,), jnp.int32) for _ in range(4)]
              for _ in range(3)] for _ in range(2)],
        r_v=[[pltpu.VMEM((_P, _FEAT), jnp.float32) for _ in range(4)]
             for _ in range(2)],
        out_v=pltpu.VMEM((_P, 3 * _FEAT), jnp.float32),
        sems=[pltpu.SemaphoreType.DMA for _ in range(2)],
    )
    run = pl.kernel(
        _tri_embed_body,
        out_type=jax.ShapeDtypeStruct((_B, 3 * _FEAT), jnp.float32),
        mesh=plsc.VectorSubcoreMesh(core_axis_name="c", subcore_axis_name="s",
                                    num_cores=_NC, num_subcores=_NS),
        scratch_types=scratch,
        compiler_params=pltpu.CompilerParams(use_tc_tiling_on_sc=False),
    )
    return run(x0, x1, x2, table_xy, table_xz, table_yz)


# speedup vs baseline: 132.9816x; 1.0695x over previous
import functools

import jax
import jax.numpy as jnp
import numpy as np
from jax import lax
from jax.experimental import pallas as pl
from jax.experimental.pallas import tpu as pltpu
from jax.experimental.pallas import tpu_sc as plsc

_RES = 512
_FEAT = 16
_MASK = _RES * _RES - 1
_PRIME1 = np.int32(np.uint32(2654435761).astype(np.int64) - (1 << 32))

_B = 1048576
_L = 16
_NC, _NS = 2, 16
_NW = _NC * _NS
_P = 512
_PW = _B // _NW
_NCHUNK = _PW // _P
_GSUB = 128

_PLANES = ((0, 1), (0, 2), (1, 2))


def _tri_embed_body(x0_hbm, x1_hbm, x2_hbm, t0_hbm, t1_hbm, t2_hbm, out_hbm,
                    x_v, w_v, i_v, r_v, out_v, sems, sem_x, sem_o):
    wid = lax.axis_index("s") * _NC + lax.axis_index("c")
    base = wid * _PW
    tables = (t0_hbm, t1_hbm, t2_hbm)

    def prefetch_x(c, ci):
        p0 = base + c * _P
        for k, xh in enumerate((x0_hbm, x1_hbm, x2_hbm)):
            pltpu.async_copy(xh.at[pl.ds(p0, _P)], x_v[ci][k], sem_x[ci])

    def hash_chunk(ci):
        for k in range(3):
            pltpu.make_async_copy(x0_hbm.at[pl.ds(0, _P)], x_v[ci][k],
                                  sem_x[ci]).wait()

        @plsc.parallel_loop(0, _P // _L, unroll=2)
        def hash_body(g):
            s = pl.ds(g * _L, _L)
            cell = []
            for k in range(3):
                f = x_v[ci][k][s] * float(_RES)
                i = f.astype(jnp.int32)
                i = jnp.minimum(jnp.maximum(i, 0), _RES - 1)
                w_v[ci][k][s] = f - i.astype(jnp.float32)
                cell.append(i)
            for p_i, (a, b) in enumerate(_PLANES):
                ia, ib = cell[a], cell[b]
                t0 = ib * _PRIME1
                t1 = t0 + _PRIME1
                i_v[ci][p_i][0][s] = (ia ^ t0) & _MASK
                i_v[ci][p_i][1][s] = (ia ^ t1) & _MASK
                ia1 = ia + 1
                i_v[ci][p_i][2][s] = (ia1 ^ t0) & _MASK
                i_v[ci][p_i][3][s] = (ia1 ^ t1) & _MASK

    def issue(ci, p_i, b):
        table = tables[p_i]
        for j in range(_P // _GSUB):
            js = pl.ds(j * _GSUB, _GSUB)
            for k in range(4):
                pltpu.async_copy(table.at[i_v[ci][p_i][k].at[js]],
                                 r_v[b][k].at[js], sems[b])

    def drain(b):
        for k in range(4):
            pltpu.make_async_copy(t0_hbm.at[pl.ds(0, _P)], r_v[b][k],
                                  sems[b]).wait()

    def interp(ci, p_i, b):
        a, bb = _PLANES[p_i]

        @plsc.parallel_loop(0, _P // _L)
        def interp_body(g):
            s = pl.ds(g * _L, _L)
            wa16 = w_v[ci][a][s]
            wb16 = w_v[ci][bb][s]
            for k in range(_L):
                p = g * _L + k
                wa = jnp.zeros((_L,), jnp.float32) + wa16[k]
                wb = jnp.zeros((_L,), jnp.float32) + wb16[k]
                e00 = r_v[b][0][p]
                e01 = r_v[b][1][p]
                e10 = r_v[b][2][p]
                e11 = r_v[b][3][p]
                lo = e00 + wa * (e10 - e00)
                hi = e01 + wa * (e11 - e01)
                out_v[p, pl.ds(p_i * _FEAT, _FEAT)] = lo + wb * (hi - lo)

    def drain_out():
        pltpu.make_async_copy(out_hbm.at[pl.ds(0, _P)], out_v, sem_o).wait()

    def unit(g, j, last=False):
        c = 2 * g + (j // 3)
        p_i = j % 3
        b = j % 2
        ci = j // 3
        if not last:
            jn = j + 1
            cn = 2 * g + (jn // 3)
            if jn % 3 == 2:
                cx = 2 * g + ((jn + 1) // 3)
                prefetch_x(jnp.minimum(cx, _NCHUNK - 1), ((jn + 1) // 3) % 2)
            if jn % 3 == 0:
                hash_chunk((jn // 3) % 2)
            issue((jn // 3) % 2, jn % 3, jn % 2)
        drain(b)
        if p_i == 0:
            drain_out()
        interp(ci, p_i, b)
        if p_i == 2:
            pltpu.async_copy(out_v, out_hbm.at[pl.ds(base + c * _P, _P)],
                             sem_o)

    prefetch_x(0, 0)
    hash_chunk(0)
    issue(0, 0, 0)
    pltpu.async_copy(out_v, out_hbm.at[pl.ds(base, _P)], sem_o)

    def group(g, carry):
        for j in range(6):
            unit(g, j)
        return carry

    lax.fori_loop(0, _NCHUNK // 2, group, 0)
    drain(0)
    drain_out()


@jax.jit
def kernel(x, table_xy, table_xz, table_yz):
    x0 = x[:, 0]
    x1 = x[:, 1]
    x2 = x[:, 2]
    scratch = dict(
        x_v=[[pltpu.VMEM((_P,), jnp.float32) for _ in range(3)]
             for _ in range(2)],
        w_v=[[pltpu.VMEM((_P,), jnp.float32) for _ in range(3)]
             for _ in range(2)],
        i_v=[[[pltpu.VMEM((_P,), jnp.int32) for _ in range(4)]
              for _ in range(3)] for _ in range(2)],
        r_v=[[pltpu.VMEM((_P, _FEAT), jnp.float32) for _ in range(4)]
             for _ in range(2)],
        out_v=pltpu.VMEM((_P, 3 * _FEAT), jnp.float32),
        sems=[pltpu.SemaphoreType.DMA for _ in range(2)],
        sem_x=[pltpu.SemaphoreType.DMA for _ in range(2)],
        sem_o=pltpu.SemaphoreType.DMA,
    )
    run = pl.kernel(
        _tri_embed_body,
        out_type=jax.ShapeDtypeStruct((_B, 3 * _FEAT), jnp.float32),
        mesh=plsc.VectorSubcoreMesh(core_axis_name="c", subcore_axis_name="s",
                                    num_cores=_NC, num_subcores=_NS),
        scratch_types=scratch,
        compiler_params=pltpu.CompilerParams(use_tc_tiling_on_sc=False),
    )
    return run(x0, x1, x2, table_xy, table_xz, table_yz)


# speedup vs baseline: 133.5195x; 1.0040x over previous
import functools

import jax
import jax.numpy as jnp
import numpy as np
from jax import lax
from jax.experimental import pallas as pl
from jax.experimental.pallas import tpu as pltpu
from jax.experimental.pallas import tpu_sc as plsc

_RES = 512
_FEAT = 16
_MASK = _RES * _RES - 1
_PRIME1 = np.int32(np.uint32(2654435761).astype(np.int64) - (1 << 32))

_B = 1048576
_L = 16
_NC, _NS = 2, 16
_NW = _NC * _NS
_P = 512
_PW = _B // _NW
_NCHUNK = _PW // _P
_GSUB = 64

_PLANES = ((0, 1), (0, 2), (1, 2))


def _tri_embed_body(x0_hbm, x1_hbm, x2_hbm, t0_hbm, t1_hbm, t2_hbm, out_hbm,
                    x_v, w_v, i_v, r_v, out_v, sems, sem_x, sem_o):
    wid = lax.axis_index("s") * _NC + lax.axis_index("c")
    base = wid * _PW
    tables = (t0_hbm, t1_hbm, t2_hbm)

    def prefetch_x(c, ci):
        p0 = base + c * _P
        for k, xh in enumerate((x0_hbm, x1_hbm, x2_hbm)):
            pltpu.async_copy(xh.at[pl.ds(p0, _P)], x_v[ci][k], sem_x[ci])

    def hash_chunk(ci):
        for k in range(3):
            pltpu.make_async_copy(x0_hbm.at[pl.ds(0, _P)], x_v[ci][k],
                                  sem_x[ci]).wait()

        @plsc.parallel_loop(0, _P // _L, unroll=2)
        def hash_body(g):
            s = pl.ds(g * _L, _L)
            cell = []
            for k in range(3):
                f = x_v[ci][k][s] * float(_RES)
                i = f.astype(jnp.int32)
                i = jnp.minimum(jnp.maximum(i, 0), _RES - 1)
                w_v[ci][k][s] = f - i.astype(jnp.float32)
                cell.append(i)
            for p_i, (a, b) in enumerate(_PLANES):
                ia, ib = cell[a], cell[b]
                t0 = ib * _PRIME1
                t1 = t0 + _PRIME1
                i_v[ci][p_i][0][s] = (ia ^ t0) & _MASK
                i_v[ci][p_i][1][s] = (ia ^ t1) & _MASK
                ia1 = ia + 1
                i_v[ci][p_i][2][s] = (ia1 ^ t0) & _MASK
                i_v[ci][p_i][3][s] = (ia1 ^ t1) & _MASK

    def issue(ci, p_i, b):
        table = tables[p_i]
        for j in range(_P // _GSUB):
            js = pl.ds(j * _GSUB, _GSUB)
            for k in range(4):
                pltpu.async_copy(table.at[i_v[ci][p_i][k].at[js]],
                                 r_v[b][k].at[js], sems[b])

    def drain(b):
        for k in range(4):
            pltpu.make_async_copy(t0_hbm.at[pl.ds(0, _P)], r_v[b][k],
                                  sems[b]).wait()

    def interp(ci, p_i, b):
        a, bb = _PLANES[p_i]

        @plsc.parallel_loop(0, _P // _L)
        def interp_body(g):
            s = pl.ds(g * _L, _L)
            wa16 = w_v[ci][a][s]
            wb16 = w_v[ci][bb][s]
            for k in range(_L):
                p = g * _L + k
                wa = jnp.zeros((_L,), jnp.float32) + wa16[k]
                wb = jnp.zeros((_L,), jnp.float32) + wb16[k]
                e00 = r_v[b][0][p]
                e01 = r_v[b][1][p]
                e10 = r_v[b][2][p]
                e11 = r_v[b][3][p]
                lo = e00 + wa * (e10 - e00)
                hi = e01 + wa * (e11 - e01)
                out_v[p, pl.ds(p_i * _FEAT, _FEAT)] = lo + wb * (hi - lo)

    def drain_out():
        pltpu.make_async_copy(out_hbm.at[pl.ds(0, _P)], out_v, sem_o).wait()

    def unit(g, j, last=False):
        c = 2 * g + (j // 3)
        p_i = j % 3
        b = j % 2
        ci = j // 3
        if not last:
            jn = j + 1
            cn = 2 * g + (jn // 3)
            if jn % 3 == 2:
                cx = 2 * g + ((jn + 1) // 3)
                prefetch_x(jnp.minimum(cx, _NCHUNK - 1), ((jn + 1) // 3) % 2)
            if jn % 3 == 0:
                hash_chunk((jn // 3) % 2)
            issue((jn // 3) % 2, jn % 3, jn % 2)
        drain(b)
        if p_i == 0:
            drain_out()
        interp(ci, p_i, b)
        if p_i == 2:
            pltpu.async_copy(out_v, out_hbm.at[pl.ds(base + c * _P, _P)],
                             sem_o)

    prefetch_x(0, 0)
    hash_chunk(0)
    issue(0, 0, 0)
    pltpu.async_copy(out_v, out_hbm.at[pl.ds(base, _P)], sem_o)

    def group(g, carry):
        for j in range(6):
            unit(g, j)
        return carry

    lax.fori_loop(0, _NCHUNK // 2, group, 0)
    drain(0)
    drain_out()


@jax.jit
def kernel(x, table_xy, table_xz, table_yz):
    x0 = x[:, 0]
    x1 = x[:, 1]
    x2 = x[:, 2]
    scratch = dict(
        x_v=[[pltpu.VMEM((_P,), jnp.float32) for _ in range(3)]
             for _ in range(2)],
        w_v=[[pltpu.VMEM((_P,), jnp.float32) for _ in range(3)]
             for _ in range(2)],
        i_v=[[[pltpu.VMEM((_P,), jnp.int32) for _ in range(4)]
              for _ in range(3)] for _ in range(2)],
        r_v=[[pltpu.VMEM((_P, _FEAT), jnp.float32) for _ in range(4)]
             for _ in range(2)],
        out_v=pltpu.VMEM((_P, 3 * _FEAT), jnp.float32),
        sems=[pltpu.SemaphoreType.DMA for _ in range(2)],
        sem_x=[pltpu.SemaphoreType.DMA for _ in range(2)],
        sem_o=pltpu.SemaphoreType.DMA,
    )
    run = pl.kernel(
        _tri_embed_body,
        out_type=jax.ShapeDtypeStruct((_B, 3 * _FEAT), jnp.float32),
        mesh=plsc.VectorSubcoreMesh(core_axis_name="c", subcore_axis_name="s",
                                    num_cores=_NC, num_subcores=_NS),
        scratch_types=scratch,
        compiler_params=pltpu.CompilerParams(use_tc_tiling_on_sc=False),
    )
    return run(x0, x1, x2, table_xy, table_xz, table_yz)
